# baseline (speedup 1.0000x reference)
import functools

import jax
import jax.numpy as jnp
from jax import lax
from jax.experimental import pallas as pl
from jax.experimental.pallas import tpu as pltpu
from jax.experimental.pallas import tpu_sc as plsc

_N = 4096
_NW = 32
_RPW = _N // _NW
_L = 16
_CHUNKS = _N // _L

_B = 512
_LO = -16.0
_HI = 16.0
_W = (_HI - _LO) / _B
_SCALE = _B / (_HI - _LO)
_PAD = 128
_SN = _N + 2 * _PAD
_STEP = 64
_BIG = 1 << 20


def _sqrt16(x):
    i = plsc.bitcast(x, jnp.int32)
    i = jnp.int32(0x5F3759DF) - (i >> 1)
    y = plsc.bitcast(i, jnp.float32)
    xh = x * jnp.float32(0.5)
    for _ in range(3):
        y = y * (jnp.float32(1.5) - xh * y * y)
    return x * y


def _knn_body(pts_hbm, scl_hbm, out_hbm,
              pts_v, bid_v, slot_v, sx_v, sy_v, sz_v, sbid_v,
              hist_v, cum_v, start_v, scl_v, s_v, out_v):
    cid = lax.axis_index("c")
    sid = lax.axis_index("s")
    wid = sid * 2 + cid
    base = wid * _RPW

    pltpu.sync_copy(pts_hbm, pts_v)
    for k in range(3):
        pltpu.sync_copy(scl_hbm.at[pl.ds(k * _N + base, _RPW)],
                        scl_v.at[pl.ds(k * _RPW, _RPW)])

    inf = jnp.float32(jnp.inf)
    lanes = lax.iota(jnp.int32, 16)
    lane0 = lanes == 0
    infv = jnp.full((_L,), inf)
    zerov = jnp.zeros((_L,), jnp.float32)
    zeroi = jnp.zeros((_L,), jnp.int32)
    onesi = jnp.ones((_L,), jnp.int32)
    lanesB = lanes * _B

    def z_body(j, carry):
        hist_v[pl.ds(j * _L, _L)] = zeroi
        return carry

    lax.fori_loop(0, (16 * _B) // _L, z_body, 0)

    def p_body(j, carry):
        lo_off = j * _L
        hi_off = _PAD + _N + j * _L
        for off, sentinel in ((lo_off, -_BIG), (hi_off, _BIG)):
            sx_v[pl.ds(off, _L)] = infv
            sy_v[pl.ds(off, _L)] = zerov
            sz_v[pl.ds(off, _L)] = zerov
            sbid_v[pl.ds(off, _L)] = jnp.full((_L,), sentinel, jnp.int32)
        return carry

    lax.fori_loop(0, _PAD // _L, p_body, 0)

    def a_body(j, carry):
        off = j * _L
        x = pts_v[pl.ds(off, _L)]
        t = (x - jnp.float32(_LO)) * jnp.float32(_SCALE)
        t = jnp.maximum(jnp.minimum(t, jnp.float32(_B - 1)), jnp.float32(0.0))
        bid = t.astype(jnp.int32)
        bid_v[pl.ds(off, _L)] = bid
        plsc.addupdate_scatter(hist_v, [lanesB + bid], onesi)
        return carry

    lax.fori_loop(0, _CHUNKS, a_body, 0)

    def c_body(j, carry):
        boff = j * _L
        run = zeroi
        for l in range(16):
            o = l * _B + boff
            h = hist_v[pl.ds(o, _L)]
            cum_v[pl.ds(o, _L)] = run
            hist_v[pl.ds(o, _L)] = zeroi
            run = run + h
        start_v[pl.ds(boff, _L)] = run
        return carry

    lax.fori_loop(0, _B // _L, c_body, 0)

    def s_body(j, tot):
        boff = j * _L
        t = start_v[pl.ds(boff, _L)]
        excl = plsc.cumsum(t) - t + jnp.full((_L,), tot, jnp.int32)
        start_v[pl.ds(boff, _L)] = excl
        return tot + jnp.sum(t)

    lax.fori_loop(0, _B // _L, s_body, jnp.int32(0))

    def b_body(j, carry):
        off = j * _L
        bid = bid_v[pl.ds(off, _L)]
        idx2 = lanesB + bid
        arr = plsc.load_gather(hist_v, [idx2])
        plsc.addupdate_scatter(hist_v, [idx2], onesi)
        cro = plsc.load_gather(cum_v, [idx2])
        stt = plsc.load_gather(start_v, [bid])
        slot = stt + cro + arr + _PAD
        plsc.store_scatter(sx_v, [slot], pts_v[pl.ds(off, _L)])
        plsc.store_scatter(sy_v, [slot], pts_v[pl.ds(_N + off, _L)])
        plsc.store_scatter(sz_v, [slot], pts_v[pl.ds(2 * _N + off, _L)])
        plsc.store_scatter(sbid_v, [slot], bid)
        slot_v[pl.ds(off, _L)] = slot
        return carry

    lax.fori_loop(0, _CHUNKS, b_body, 0)

    def _insert3(a, b, c, d):
        a2 = jnp.minimum(a, d)
        t = jnp.maximum(a, d)
        b2 = jnp.minimum(b, t)
        t = jnp.maximum(b, t)
        c2 = jnp.minimum(c, t)
        return a2, b2, c2

    def _scan_chunk(off, qx, qy, qz, a, b, c):
        dx = sx_v[pl.ds(off, _L)] - qx
        d = dx * dx
        dy = sy_v[pl.ds(off, _L)] - qy
        d = d + dy * dy
        dz = sz_v[pl.ds(off, _L)] - qz
        d = d + dz * dz
        return _insert3(a, b, c, d)

    def q_body(q, carry_q):
        qiv = jnp.full((_L,), base + q, jnp.int32)
        qx = plsc.load_gather(pts_v, [qiv])
        qy = plsc.load_gather(pts_v, [qiv + _N])
        qz = plsc.load_gather(pts_v, [qiv + 2 * _N])
        bqv = plsc.load_gather(bid_v, [qiv])
        slotq = plsc.load_gather(slot_v, [qiv])
        p0 = jnp.min(plsc.load_gather(start_v, [bqv])) + _PAD

        plsc.store_scatter(sx_v, [slotq], infv, mask=lane0)

        def r_cond(carry):
            return carry[1] > 0

        def r_body(carry):
            p, _, a, b, c = carry
            for u in range(_STEP // _L):
                a, b, c = _scan_chunk(p + u * _L, qx, qy, qz, a, b, c)
            pn = p + _STEP
            bv = plsc.load_gather(sbid_v, [jnp.full((_L,), pn, jnp.int32)])
            xb = jnp.where(bv == 0, jnp.float32(-3e38),
                           bv.astype(jnp.float32) * jnp.float32(_W)
                           + jnp.float32(_LO))
            gap = xb - qx
            g = jnp.min(gap)
            m = jnp.min(c)
            stop = jnp.logical_or(
                jnp.logical_and(g > 0, g * g > m), pn >= _PAD + _N)
            return (pn, jnp.where(stop, 0, 1), a, b, c)

        init_r = (p0, jnp.int32(1), infv, infv, infv)
        _, _, a, b, c = lax.while_loop(r_cond, r_body, init_r)

        def l_body(carry):
            p, _, a, b, c = carry
            for u in range(_STEP // _L):
                a, b, c = _scan_chunk(p - (u + 1) * _L, qx, qy, qz, a, b, c)
            pn = p - _STEP
            bv = plsc.load_gather(sbid_v,
                                  [jnp.full((_L,), pn - 1, jnp.int32)])
            xb = jnp.where(bv == _B - 1, jnp.float32(3e38),
                           (bv + 1).astype(jnp.float32) * jnp.float32(_W)
                           + jnp.float32(_LO))
            gap = qx - xb
            g = jnp.min(gap)
            m = jnp.min(c)
            stop = jnp.logical_or(
                jnp.logical_and(g > 0, g * g > m), pn <= _PAD)
            return (pn, jnp.where(stop, 0, 1), a, b, c)

        init_l = (p0, jnp.int32(1), a, b, c)
        _, _, a, b, c = lax.while_loop(r_cond, l_body, init_l)

        plsc.store_scatter(sx_v, [slotq], qx, mask=lane0)

        qv = jnp.full((_L,), q, jnp.int32)
        for r in range(3):
            m = jnp.min(a)
            plsc.store_scatter(s_v, [qv + r * _RPW],
                               jnp.full((_L,), m, jnp.float32),
                               mask=lane0)
            sel = lanes == plsc.all_reduce_ffs(a == m)
            a = jnp.where(sel, b, a)
            b = jnp.where(sel, c, b)
            c = jnp.where(sel, inf, c)
        return carry_q

    lax.fori_loop(0, _RPW, q_body, 0)

    third = jnp.float32(1.0 / 3.0)
    zero = jnp.float32(0.0)

    def f_body(v, carry_f):
        o = v * _L
        r = (_sqrt16(jnp.maximum(s_v[pl.ds(o, _L)], zero))
             + _sqrt16(jnp.maximum(s_v[pl.ds(_RPW + o, _L)], zero))
             + _sqrt16(jnp.maximum(s_v[pl.ds(2 * _RPW + o, _L)], zero))) * third
        r = jnp.maximum(r, jnp.float32(1e-5))
        for k in range(3):
            out_v[pl.ds(k * _RPW + o, _L)] = scl_v[pl.ds(k * _RPW + o, _L)] * r
        return carry_f

    lax.fori_loop(0, _RPW // _L, f_body, 0)

    for k in range(3):
        pltpu.sync_copy(out_v.at[pl.ds(k * _RPW, _RPW)],
                        out_hbm.at[pl.ds(k * _N + base, _RPW)])


_knn = functools.partial(
    pl.kernel,
    mesh=plsc.VectorSubcoreMesh(core_axis_name="c", subcore_axis_name="s"),
    compiler_params=pltpu.CompilerParams(needs_layout_passes=False),
    out_type=jax.ShapeDtypeStruct((3 * _N,), jnp.float32),
    scratch_types=[
        pltpu.VMEM((3 * _N,), jnp.float32),
        pltpu.VMEM((_N,), jnp.int32),
        pltpu.VMEM((_N,), jnp.int32),
        pltpu.VMEM((_SN,), jnp.float32),
        pltpu.VMEM((_SN,), jnp.float32),
        pltpu.VMEM((_SN,), jnp.float32),
        pltpu.VMEM((_SN,), jnp.int32),
        pltpu.VMEM((16 * _B,), jnp.int32),
        pltpu.VMEM((16 * _B,), jnp.int32),
        pltpu.VMEM((_B,), jnp.int32),
        pltpu.VMEM((3 * _RPW,), jnp.float32),
        pltpu.VMEM((3 * _RPW,), jnp.float32),
        pltpu.VMEM((3 * _RPW,), jnp.float32),
    ],
)(_knn_body)


def kernel(points, scales):
    out_flat = _knn(points.T.reshape(-1), scales.T.reshape(-1))
    return out_flat.reshape(3, _N).T


# speedup vs baseline: 1.5474x; 1.5474x over previous
import functools

import jax
import jax.numpy as jnp
from jax import lax
from jax.experimental import pallas as pl
from jax.experimental.pallas import tpu as pltpu
from jax.experimental.pallas import tpu_sc as plsc

_N = 4096
_NW = 32
_RPW = _N // _NW
_L = 16
_CHUNKS = _N // _L

_B = 512
_LO = -16.0
_HI = 16.0
_W = (_HI - _LO) / _B
_SCALE = _B / (_HI - _LO)
_PAD = 128
_SN = _N + 2 * _PAD
_STEP = 128
_BIG = 1 << 20


def _sqrt16(x):
    i = plsc.bitcast(x, jnp.int32)
    i = jnp.int32(0x5F3759DF) - (i >> 1)
    y = plsc.bitcast(i, jnp.float32)
    xh = x * jnp.float32(0.5)
    for _ in range(3):
        y = y * (jnp.float32(1.5) - xh * y * y)
    return x * y


def _knn_body(pts_hbm, scl_hbm, out_hbm,
              pts_v, bid_v, slot_v, sx_v, sy_v, sz_v, sbid_v,
              hist_v, cum_v, start_v, scl_v, s_v, out_v):
    cid = lax.axis_index("c")
    sid = lax.axis_index("s")
    wid = sid * 2 + cid
    base = wid * _RPW

    pltpu.sync_copy(pts_hbm, pts_v)
    for k in range(3):
        pltpu.sync_copy(scl_hbm.at[pl.ds(k * _N + base, _RPW)],
                        scl_v.at[pl.ds(k * _RPW, _RPW)])

    inf = jnp.float32(jnp.inf)
    lanes = lax.iota(jnp.int32, 16)
    lane0 = lanes == 0
    infv = jnp.full((_L,), inf)
    zerov = jnp.zeros((_L,), jnp.float32)
    zeroi = jnp.zeros((_L,), jnp.int32)
    onesi = jnp.ones((_L,), jnp.int32)
    lanesB = lanes * _B

    def z_body(j, carry):
        hist_v[pl.ds(j * _L, _L)] = zeroi
        return carry

    lax.fori_loop(0, (16 * _B) // _L, z_body, 0)

    def p_body(j, carry):
        lo_off = j * _L
        hi_off = _PAD + _N + j * _L
        for off, sentinel in ((lo_off, -_BIG), (hi_off, _BIG)):
            sx_v[pl.ds(off, _L)] = infv
            sy_v[pl.ds(off, _L)] = zerov
            sz_v[pl.ds(off, _L)] = zerov
            sbid_v[pl.ds(off, _L)] = jnp.full((_L,), sentinel, jnp.int32)
        return carry

    lax.fori_loop(0, _PAD // _L, p_body, 0)

    def a_body(j, carry):
        off = j * _L
        x = pts_v[pl.ds(off, _L)]
        t = (x - jnp.float32(_LO)) * jnp.float32(_SCALE)
        t = jnp.maximum(jnp.minimum(t, jnp.float32(_B - 1)), jnp.float32(0.0))
        bid = t.astype(jnp.int32)
        bid_v[pl.ds(off, _L)] = bid
        plsc.addupdate_scatter(hist_v, [lanesB + bid], onesi)
        return carry

    lax.fori_loop(0, _CHUNKS, a_body, 0)

    def c_body(j, carry):
        boff = j * _L
        run = zeroi
        for l in range(16):
            o = l * _B + boff
            h = hist_v[pl.ds(o, _L)]
            cum_v[pl.ds(o, _L)] = run
            hist_v[pl.ds(o, _L)] = zeroi
            run = run + h
        start_v[pl.ds(boff, _L)] = run
        return carry

    lax.fori_loop(0, _B // _L, c_body, 0)

    def s_body(j, tot):
        boff = j * _L
        t = start_v[pl.ds(boff, _L)]
        excl = plsc.cumsum(t) - t + jnp.full((_L,), tot, jnp.int32)
        start_v[pl.ds(boff, _L)] = excl
        return tot + jnp.sum(t)

    lax.fori_loop(0, _B // _L, s_body, jnp.int32(0))

    def b_body(j, carry):
        off = j * _L
        bid = bid_v[pl.ds(off, _L)]
        idx2 = lanesB + bid
        arr = plsc.load_gather(hist_v, [idx2])
        plsc.addupdate_scatter(hist_v, [idx2], onesi)
        cro = plsc.load_gather(cum_v, [idx2])
        stt = plsc.load_gather(start_v, [bid])
        slot = stt + cro + arr + _PAD
        plsc.store_scatter(sx_v, [slot], pts_v[pl.ds(off, _L)])
        plsc.store_scatter(sy_v, [slot], pts_v[pl.ds(_N + off, _L)])
        plsc.store_scatter(sz_v, [slot], pts_v[pl.ds(2 * _N + off, _L)])
        plsc.store_scatter(sbid_v, [slot], bid)
        slot_v[pl.ds(off, _L)] = slot
        return carry

    lax.fori_loop(0, _CHUNKS, b_body, 0)

    def _insert3(a, b, c, d):
        a2 = jnp.minimum(a, d)
        t = jnp.maximum(a, d)
        b2 = jnp.minimum(b, t)
        t = jnp.maximum(b, t)
        c2 = jnp.minimum(c, t)
        return a2, b2, c2

    def _scan_chunk(off, qx, qy, qz, a, b, c):
        dx = sx_v[pl.ds(off, _L)] - qx
        d = dx * dx
        dy = sy_v[pl.ds(off, _L)] - qy
        d = d + dy * dy
        dz = sz_v[pl.ds(off, _L)] - qz
        d = d + dz * dz
        return _insert3(a, b, c, d)

    def q_body(q, carry_q):
        qiv = jnp.full((_L,), base + q, jnp.int32)
        qx = plsc.load_gather(pts_v, [qiv])
        qy = plsc.load_gather(pts_v, [qiv + _N])
        qz = plsc.load_gather(pts_v, [qiv + 2 * _N])
        bqv = plsc.load_gather(bid_v, [qiv])
        slotq = plsc.load_gather(slot_v, [qiv])
        p0 = jnp.min(plsc.load_gather(start_v, [bqv])) + _PAD

        plsc.store_scatter(sx_v, [slotq], infv, mask=lane0)

        def _geo_count(gap, a, b, c):
            g2 = gap * gap
            ok = gap > jnp.float32(0.0)
            cnt = (jnp.logical_and(ok, a < g2).astype(jnp.int32)
                   + jnp.logical_and(ok, b < g2).astype(jnp.int32)
                   + jnp.logical_and(ok, c < g2).astype(jnp.int32))
            return jnp.sum(cnt) >= 3

        def r_cond(carry):
            return carry[1] > 0

        def r_body(carry):
            p, _, a, b, c = carry
            bv = plsc.load_gather(sbid_v, [jnp.full((_L,), p, jnp.int32)])
            xb = jnp.where(bv == 0, jnp.float32(-3e38),
                           bv.astype(jnp.float32) * jnp.float32(_W)
                           + jnp.float32(_LO))
            stop = jnp.logical_or(_geo_count(xb - qx, a, b, c),
                                  p + _STEP >= _PAD + _N)
            for u in range(_STEP // _L):
                a, b, c = _scan_chunk(p + u * _L, qx, qy, qz, a, b, c)
            return (p + _STEP, jnp.where(stop, jnp.int32(0), jnp.int32(1)),
                    a, b, c)

        init_r = (p0, jnp.int32(1), infv, infv, infv)
        _, _, a, b, c = lax.while_loop(r_cond, r_body, init_r)

        def l_body(carry):
            p, _, a, b, c = carry
            bv = plsc.load_gather(sbid_v,
                                  [jnp.full((_L,), p - 1, jnp.int32)])
            xb = jnp.where(bv == _B - 1, jnp.float32(3e38),
                           (bv + 1).astype(jnp.float32) * jnp.float32(_W)
                           + jnp.float32(_LO))
            stop = jnp.logical_or(_geo_count(qx - xb, a, b, c),
                                  p - _STEP <= _PAD)
            for u in range(_STEP // _L):
                a, b, c = _scan_chunk(p - (u + 1) * _L, qx, qy, qz, a, b, c)
            return (p - _STEP, jnp.where(stop, jnp.int32(0), jnp.int32(1)),
                    a, b, c)

        init_l = (p0, jnp.int32(1), a, b, c)
        _, _, a, b, c = lax.while_loop(r_cond, l_body, init_l)

        plsc.store_scatter(sx_v, [slotq], qx, mask=lane0)

        qv = jnp.full((_L,), q, jnp.int32)
        for r in range(3):
            m = jnp.min(a)
            plsc.store_scatter(s_v, [qv + r * _RPW],
                               jnp.full((_L,), m, jnp.float32),
                               mask=lane0)
            sel = lanes == plsc.all_reduce_ffs(a == m)
            a = jnp.where(sel, b, a)
            b = jnp.where(sel, c, b)
            c = jnp.where(sel, inf, c)
        return carry_q

    lax.fori_loop(0, _RPW, q_body, 0)

    third = jnp.float32(1.0 / 3.0)
    zero = jnp.float32(0.0)

    def f_body(v, carry_f):
        o = v * _L
        r = (_sqrt16(jnp.maximum(s_v[pl.ds(o, _L)], zero))
             + _sqrt16(jnp.maximum(s_v[pl.ds(_RPW + o, _L)], zero))
             + _sqrt16(jnp.maximum(s_v[pl.ds(2 * _RPW + o, _L)], zero))) * third
        r = jnp.maximum(r, jnp.float32(1e-5))
        for k in range(3):
            out_v[pl.ds(k * _RPW + o, _L)] = scl_v[pl.ds(k * _RPW + o, _L)] * r
        return carry_f

    lax.fori_loop(0, _RPW // _L, f_body, 0)

    for k in range(3):
        pltpu.sync_copy(out_v.at[pl.ds(k * _RPW, _RPW)],
                        out_hbm.at[pl.ds(k * _N + base, _RPW)])


_knn = functools.partial(
    pl.kernel,
    mesh=plsc.VectorSubcoreMesh(core_axis_name="c", subcore_axis_name="s"),
    compiler_params=pltpu.CompilerParams(needs_layout_passes=False),
    out_type=jax.ShapeDtypeStruct((3 * _N,), jnp.float32),
    scratch_types=[
        pltpu.VMEM((3 * _N,), jnp.float32),
        pltpu.VMEM((_N,), jnp.int32),
        pltpu.VMEM((_N,), jnp.int32),
        pltpu.VMEM((_SN,), jnp.float32),
        pltpu.VMEM((_SN,), jnp.float32),
        pltpu.VMEM((_SN,), jnp.float32),
        pltpu.VMEM((_SN,), jnp.int32),
        pltpu.VMEM((16 * _B,), jnp.int32),
        pltpu.VMEM((16 * _B,), jnp.int32),
        pltpu.VMEM((_B,), jnp.int32),
        pltpu.VMEM((3 * _RPW,), jnp.float32),
        pltpu.VMEM((3 * _RPW,), jnp.float32),
        pltpu.VMEM((3 * _RPW,), jnp.float32),
    ],
)(_knn_body)


def kernel(points, scales):
    out_flat = _knn(points.T.reshape(-1), scales.T.reshape(-1))
    return out_flat.reshape(3, _N).T
